# baseline (speedup 1.0000x reference)
import jax
import jax.numpy as jnp
from jax.experimental import pallas as pl

_N_CLASSES = 1000
_BLOCK_ROWS = 512


def _onehot_block(x_ref, o_ref):
    classes = jax.lax.broadcasted_iota(jnp.int32, (_BLOCK_ROWS, _N_CLASSES), 1)
    o_ref[...] = (x_ref[...] == classes).astype(jnp.int32)


def kernel(x):
    b, s = x.shape
    rows = b * s
    xf = x.reshape(rows, 1)
    out = pl.pallas_call(
        _onehot_block,
        grid=(rows // _BLOCK_ROWS,),
        in_specs=[pl.BlockSpec((_BLOCK_ROWS, 1), lambda i: (i, 0))],
        out_specs=pl.BlockSpec((_BLOCK_ROWS, _N_CLASSES), lambda i: (i, 0)),
        out_shape=jax.ShapeDtypeStruct((rows, _N_CLASSES), jnp.int32),
    )(xf)
    return out.reshape(b, s, _N_CLASSES)


# speedup vs baseline: 1.3938x; 1.3938x over previous
import jax
import jax.numpy as jnp
from jax.experimental import pallas as pl

_N_CLASSES = 1000
_BLOCK_B = 8


def _onehot_block(x_ref, o_ref):
    s = x_ref.shape[1]
    classes = jax.lax.broadcasted_iota(jnp.int32, (_BLOCK_B, s, _N_CLASSES), 2)
    o_ref[...] = (x_ref[...][..., None] == classes).astype(jnp.int32)


def kernel(x):
    b, s = x.shape
    return pl.pallas_call(
        _onehot_block,
        grid=(b // _BLOCK_B,),
        in_specs=[pl.BlockSpec((_BLOCK_B, s), lambda i: (i, 0))],
        out_specs=pl.BlockSpec((_BLOCK_B, s, _N_CLASSES), lambda i: (i, 0, 0)),
        out_shape=jax.ShapeDtypeStruct((b, s, _N_CLASSES), jnp.int32),
    )(x)


# speedup vs baseline: 1.5619x; 1.1206x over previous
import jax
import jax.numpy as jnp
from jax.experimental import pallas as pl

_N_CLASSES = 1000
_BLOCK_B = 32


def _onehot_block(x_ref, o_ref):
    s = x_ref.shape[1]
    classes = jax.lax.broadcasted_iota(jnp.int32, (_BLOCK_B, s, _N_CLASSES), 2)
    o_ref[...] = (x_ref[...][..., None] == classes).astype(jnp.int32)


def kernel(x):
    b, s = x.shape
    return pl.pallas_call(
        _onehot_block,
        grid=(b // _BLOCK_B,),
        in_specs=[pl.BlockSpec((_BLOCK_B, s), lambda i: (i, 0))],
        out_specs=pl.BlockSpec((_BLOCK_B, s, _N_CLASSES), lambda i: (i, 0, 0)),
        out_shape=jax.ShapeDtypeStruct((b, s, _N_CLASSES), jnp.int32),
    )(x)
